# baseline (speedup 1.0000x reference)
import functools

import jax
import jax.numpy as jnp
from jax import lax
from jax.experimental import pallas as pl
from jax.experimental.pallas import tpu as pltpu
from jax.experimental.pallas import tpu_sc as plsc

_BATCH = 16384
_D_STATE = 512
_D_OUT = 512
_EMBED = 16

_NW = 32
_BPW = _BATCH // _NW
_G = 16
_NG = _BPW // _G
_NBUF = 3

_TB = 1024
_CB = 4096


def _sc_gather_t(table_t, idx):
    mesh = plsc.VectorSubcoreMesh(core_axis_name="c", subcore_axis_name="s")

    @functools.partial(
        pl.kernel,
        out_type=jax.ShapeDtypeStruct((_EMBED, _BATCH), table_t.dtype),
        mesh=mesh,
        compiler_params=pltpu.CompilerParams(
            use_tc_tiling_on_sc=True, needs_layout_passes=False
        ),
        scratch_types=[
            pltpu.VMEM((_BPW,), jnp.int32),
            pltpu.VMEM((_NBUF, _EMBED, _G * 128), jnp.float32),
            pltpu.VMEM((_EMBED, _BPW), jnp.float32),
            pltpu.SemaphoreType.DMA,
            pltpu.SemaphoreType.DMA,
            pltpu.SemaphoreType.DMA,
            pltpu.SemaphoreType.DMA,
        ],
    )
    def run(tab_hbm, idx_hbm, out_hbm, idx_v, slabs, acc,
            sem0, sem1, sem2, osem):
        wid = lax.axis_index("s") * 2 + lax.axis_index("c")
        base = wid * _BPW
        pltpu.async_copy(idx_hbm.at[pl.ds(base, _BPW)], idx_v, sem0).wait()

        sems = (sem0, sem1, sem2)
        lane_iota = lax.iota(jnp.int32, 16)

        def fetch_group(g, buf):
            v = idx_v[pl.ds(g * _G, _G)]
            for k in range(_G):
                lane0 = pl.multiple_of((v[k] >> 7) << 7, 128)
                pltpu.make_async_copy(
                    tab_hbm.at[:, pl.ds(lane0, 128)],
                    slabs.at[buf, :, pl.ds(k * 128, 128)],
                    sems[buf],
                ).start()

        def drain_group(buf):
            pltpu.make_async_copy(
                tab_hbm.at[:, pl.ds(0, _G * 128)],
                slabs.at[buf],
                sems[buf],
            ).wait()

        def extract_group(g, buf):
            v = idx_v[pl.ds(g * _G, _G)]
            for k in range(_G):
                lane = jnp.full((16,), k * 128 + (v[k] & 127), jnp.int32)
                vals = plsc.load_gather(slabs.at[buf], [lane_iota, lane])
                col = jnp.full((16,), g * _G + k, jnp.int32)
                plsc.store_scatter(acc, [lane_iota, col], vals)

        for b in range(_NBUF - 1):
            fetch_group(b, b)

        @pl.loop(0, _NG - 2, step=_NBUF)
        def _(g):
            for b in range(_NBUF):
                fetch_group(g + b + _NBUF - 1, (b + _NBUF - 1) % _NBUF)
                drain_group(b)
                extract_group(g + b, b)

        for b in range(2):
            drain_group(b)
            extract_group(_NG - 2 + b, b)

        pltpu.async_copy(acc, out_hbm.at[:, pl.ds(base, _BPW)], osem).wait()

    return run(table_t, idx)


def _tc_matmul_rnn(state, W_enc, rnn_hxs):
    def body(s_ref, w_ref, r_ref, o_ref, r_out_ref):
        s = s_ref[...].astype(jnp.bfloat16)
        w = w_ref[...].astype(jnp.bfloat16)
        o_ref[...] = lax.dot_general(
            w, s, (((0,), (1,)), ((), ())),
            preferred_element_type=jnp.float32,
        )
        r_out_ref[...] = r_ref[...]

    return pl.pallas_call(
        body,
        grid=(_BATCH // _TB,),
        in_specs=[
            pl.BlockSpec((_TB, _D_STATE), lambda i: (i, 0)),
            pl.BlockSpec((_D_STATE, _D_OUT), lambda i: (0, 0)),
            pl.BlockSpec((_TB, _D_OUT), lambda i: (i, 0)),
        ],
        out_specs=[
            pl.BlockSpec((_D_OUT, _TB), lambda i: (0, i)),
            pl.BlockSpec((_TB, _D_OUT), lambda i: (i, 0)),
        ],
        out_shape=[
            jax.ShapeDtypeStruct((_D_OUT + _EMBED, _BATCH), jnp.float32),
            jax.ShapeDtypeStruct((_BATCH, _D_OUT), jnp.float32),
        ],
        compiler_params=pltpu.CompilerParams(
            dimension_semantics=("parallel",)
        ),
    )(state, W_enc, rnn_hxs)


def _tc_concat_act(out_partial, act_t):
    def body(_, a_ref, o_ref):
        o_ref[...] = a_ref[...]

    return pl.pallas_call(
        body,
        grid=(_BATCH // _CB,),
        in_specs=[
            pl.BlockSpec(memory_space=pl.ANY),
            pl.BlockSpec((_EMBED, _CB), lambda i: (0, i)),
        ],
        out_specs=pl.BlockSpec(
            (_EMBED, _CB), lambda i: (_D_OUT // _EMBED, i)
        ),
        out_shape=jax.ShapeDtypeStruct((_D_OUT + _EMBED, _BATCH), jnp.float32),
        input_output_aliases={0: 0},
    )(out_partial, act_t)


def kernel(state, last_action, rnn_hxs, W_enc, table):
    idx = last_action.astype(jnp.int32)
    act_t = _sc_gather_t(table.T, idx)
    out_partial, rnn_out = _tc_matmul_rnn(state, W_enc, rnn_hxs)
    out_t = _tc_concat_act(out_partial, act_t)
    return out_t.T, rnn_out


# speedup vs baseline: 1.0129x; 1.0129x over previous
import functools

import jax
import jax.numpy as jnp
from jax import lax
from jax.experimental import pallas as pl
from jax.experimental.pallas import tpu as pltpu
from jax.experimental.pallas import tpu_sc as plsc

_BATCH = 16384
_D_STATE = 512
_D_OUT = 512
_EMBED = 16

_NW = 32
_BPW = _BATCH // _NW
_G = 16
_NG = _BPW // _G
_NBUF = 3

_TB = 1024
_CB = 4096


def _sc_gather_t(table_t, idx):
    mesh = plsc.VectorSubcoreMesh(core_axis_name="c", subcore_axis_name="s")

    @functools.partial(
        pl.kernel,
        out_type=jax.ShapeDtypeStruct((_EMBED, _BATCH), table_t.dtype),
        mesh=mesh,
        compiler_params=pltpu.CompilerParams(
            use_tc_tiling_on_sc=True, needs_layout_passes=False
        ),
        scratch_types=[
            pltpu.VMEM((_BPW,), jnp.int32),
            pltpu.VMEM((_NBUF, _EMBED, _G * 128), jnp.float32),
            pltpu.VMEM((_EMBED, _BPW), jnp.float32),
            pltpu.SemaphoreType.DMA,
            pltpu.SemaphoreType.DMA,
            pltpu.SemaphoreType.DMA,
            pltpu.SemaphoreType.DMA,
        ],
    )
    def run(tab_hbm, idx_hbm, out_hbm, idx_v, slabs, acc,
            sem0, sem1, sem2, osem):
        wid = lax.axis_index("s") * 2 + lax.axis_index("c")
        base = wid * _BPW
        pltpu.async_copy(idx_hbm.at[pl.ds(base, _BPW)], idx_v, sem0).wait()

        sems = (sem0, sem1, sem2)
        lane_iota = lax.iota(jnp.int32, 16)

        def fetch_group(g, buf):
            v = idx_v[pl.ds(g * _G, _G)]
            for k in range(_G):
                lane0 = pl.multiple_of((v[k] >> 7) << 7, 128)
                pltpu.make_async_copy(
                    tab_hbm.at[:, pl.ds(lane0, 128)],
                    slabs.at[buf, :, pl.ds(k * 128, 128)],
                    sems[buf],
                ).start()

        def drain_group(buf):
            pltpu.make_async_copy(
                tab_hbm.at[:, pl.ds(0, _G * 128)],
                slabs.at[buf],
                sems[buf],
            ).wait()

        def extract_group(g, buf):
            v = idx_v[pl.ds(g * _G, _G)]
            src_cols = lane_iota * 128 + (v & 127)
            dst_cols = lane_iota + g * _G
            for e in range(_EMBED):
                row = jnp.full((16,), e, jnp.int32)
                vals = plsc.load_gather(slabs.at[buf], [row, src_cols])
                plsc.store_scatter(acc, [row, dst_cols], vals)

        for b in range(_NBUF - 1):
            fetch_group(b, b)

        @pl.loop(0, _NG - 2, step=_NBUF)
        def _(g):
            for b in range(_NBUF):
                fetch_group(g + b + _NBUF - 1, (b + _NBUF - 1) % _NBUF)
                drain_group(b)
                extract_group(g + b, b)

        for b in range(2):
            drain_group(b)
            extract_group(_NG - 2 + b, b)

        pltpu.async_copy(acc, out_hbm.at[:, pl.ds(base, _BPW)], osem).wait()

    return run(table_t, idx)


def _tc_matmul_rnn(state, W_enc, rnn_hxs):
    def body(s_ref, w_ref, r_ref, o_ref, r_out_ref):
        s = s_ref[...].astype(jnp.bfloat16)
        w = w_ref[...].astype(jnp.bfloat16)
        o_ref[...] = lax.dot_general(
            w, s, (((0,), (1,)), ((), ())),
            preferred_element_type=jnp.float32,
        )
        r_out_ref[...] = r_ref[...]

    return pl.pallas_call(
        body,
        grid=(_BATCH // _TB,),
        in_specs=[
            pl.BlockSpec((_TB, _D_STATE), lambda i: (i, 0)),
            pl.BlockSpec((_D_STATE, _D_OUT), lambda i: (0, 0)),
            pl.BlockSpec((_TB, _D_OUT), lambda i: (i, 0)),
        ],
        out_specs=[
            pl.BlockSpec((_D_OUT, _TB), lambda i: (0, i)),
            pl.BlockSpec((_TB, _D_OUT), lambda i: (i, 0)),
        ],
        out_shape=[
            jax.ShapeDtypeStruct((_D_OUT + _EMBED, _BATCH), jnp.float32),
            jax.ShapeDtypeStruct((_BATCH, _D_OUT), jnp.float32),
        ],
        compiler_params=pltpu.CompilerParams(
            dimension_semantics=("parallel",)
        ),
    )(state, W_enc, rnn_hxs)


def _tc_concat_act(out_partial, act_t):
    def body(_, a_ref, o_ref):
        o_ref[...] = a_ref[...]

    return pl.pallas_call(
        body,
        grid=(_BATCH // _CB,),
        in_specs=[
            pl.BlockSpec(memory_space=pl.ANY),
            pl.BlockSpec((_EMBED, _CB), lambda i: (0, i)),
        ],
        out_specs=pl.BlockSpec(
            (_EMBED, _CB), lambda i: (_D_OUT // _EMBED, i)
        ),
        out_shape=jax.ShapeDtypeStruct((_D_OUT + _EMBED, _BATCH), jnp.float32),
        input_output_aliases={0: 0},
    )(out_partial, act_t)


def kernel(state, last_action, rnn_hxs, W_enc, table):
    idx = last_action.astype(jnp.int32)
    act_t = _sc_gather_t(table.T, idx)
    out_partial, rnn_out = _tc_matmul_rnn(state, W_enc, rnn_hxs)
    out_t = _tc_concat_act(out_partial, act_t)
    return out_t.T, rnn_out
